# Initial kernel scaffold; baseline (speedup 1.0000x reference)
import jax
import jax.numpy as jnp
from jax.experimental import pallas as pl


def kernel(x, duty_cycles, boost_strength):
    raise NotImplementedError("write your pallas kernel here")


# speedup vs baseline: 57.0635x; 57.0635x over previous
import jax
import jax.numpy as jnp
from jax.experimental import pallas as pl
from jax.experimental.pallas import tpu as pltpu

B, H, W, C = 8, 128, 128, 96
HW = H * W
N = H * W * C
K = int(0.1 * N)
TD = float(K) / float(N)


def _tc_body(x_ref, dc_ref, bs_ref, o_ref):
    bs = jnp.maximum(bs_ref[0], 0.0)
    bf = jnp.exp((jnp.float32(TD) - dc_ref[...]) * bs)
    xb = x_ref[0]
    boosted = xb * bf
    bits = jax.lax.bitcast_convert_type(boosted, jnp.int32)
    key = jnp.where(bits < 0, bits ^ jnp.int32(0x7FFFFFFF), bits)
    minint = jnp.int32(-2147483648)

    def step(i, p):
        b = 31 - i
        cand = p | (jnp.int32(1) << b)
        thr = cand ^ minint
        cnt = jnp.sum((key >= thr).astype(jnp.int32))
        return jnp.where(cnt >= K, cand, p)

    p = jax.lax.fori_loop(0, 32, step, jnp.int32(0))
    thr = p ^ minint
    o_ref[0] = jnp.where(key >= thr, xb, jnp.float32(0.0))


def kernel(x, duty_cycles, boost_strength):
    xr = x.reshape(B, HW, C)
    dc = duty_cycles.reshape(1, C)
    bs = boost_strength.reshape(1)
    out = pl.pallas_call(
        _tc_body,
        grid=(B,),
        in_specs=[
            pl.BlockSpec((1, HW, C), lambda b: (b, 0, 0)),
            pl.BlockSpec((1, C), lambda b: (0, 0)),
            pl.BlockSpec(memory_space=pltpu.SMEM),
        ],
        out_specs=pl.BlockSpec((1, HW, C), lambda b: (b, 0, 0)),
        out_shape=jax.ShapeDtypeStruct((B, HW, C), jnp.float32),
        compiler_params=pltpu.CompilerParams(
            dimension_semantics=("arbitrary",)
        ),
    )(xr, dc, bs)
    return out.reshape(B, H, W, C)
